# Initial kernel scaffold; baseline (speedup 1.0000x reference)
import jax
import jax.numpy as jnp
from jax.experimental import pallas as pl


def kernel(feats, edge_index, etype, emb0, emb1, emb2, emb3, W_rel, W_self, fcn_W1, fcn_b1, fcn_W2, fcn_b2):
    raise NotImplementedError("write your pallas kernel here")


# speedup vs baseline: 2.1580x; 2.1580x over previous
import functools

import jax
import jax.numpy as jnp
from jax import lax
from jax.experimental import pallas as pl
from jax.experimental.pallas import tpu as pltpu
from jax.experimental.pallas import tpu_sc as plsc

N = 50000
E = 800000
R = 8
D = 64
DH = 32
NL = 4
BATCH = 16
NPG = 3125

NC = 2
NS = 16
CHUNK = 128

G = (E + NS * CHUNK - 1) // (NS * CHUNK)
EPT = G * CHUNK
EPAD = NS * EPT
KB = 23
NBLK = G // KB

NW = NC * NS
GT = (N + NW * CHUNK - 1) // (NW * CHUNK)
NPT = GT * CHUNK
NPAD = NW * NPT

ACC_ROWS = NS * ((N + NS - 1) // NS + 1)
ZROWS = ACC_ROWS // NS
OPT = N // NS

BN = 2000
NB = N // BN

_f32 = jnp.float32
_i32 = jnp.int32


def _sc_embed_body(fidx, e0, e1, e2, e3, h0, h1, h2, h3, idx_v, rows_v, sem):
    c = lax.axis_index("c")
    s = lax.axis_index("s")
    w = s * NC + c
    for t, (et, ht) in enumerate(zip((e0, e1, e2, e3), (h0, h1, h2, h3))):
        pltpu.sync_copy(fidx.at[t, w], idx_v)
        for g in range(GT):
            pltpu.async_copy(et.at[idx_v.at[g]], rows_v, sem).wait()
            pltpu.sync_copy(
                rows_v, ht.at[pl.ds(w * NPT + g * CHUNK, CHUNK)])


def _sc_edges_body(rel_lo, rel_hi, flat_i, dst_i, zeros_hbm, agg_lo, agg_hi,
                   flat_v, dst_v, rows_v, acc, sem):
    c = lax.axis_index("c")
    s = lax.axis_index("s")
    pltpu.sync_copy(zeros_hbm, acc.at[pl.ds(s * ZROWS, ZROWS)])
    plsc.subcore_barrier()

    def run(rel):
        def outer(b, carry):
            pltpu.sync_copy(flat_i.at[s, pl.ds(b * KB, KB)], flat_v)
            pltpu.sync_copy(dst_i.at[s, pl.ds(b * KB, KB)], dst_v)

            def inner(j, c2):
                pltpu.async_copy(rel.at[flat_v.at[j]], rows_v, sem).wait()
                pltpu.sync_copy(rows_v, acc.at[dst_v.at[j]], add=True)
                return c2
            lax.fori_loop(0, KB, inner, 0)
            return carry
        lax.fori_loop(0, NBLK, outer, 0)

    @pl.when(c == 0)
    def _():
        run(rel_lo)

    @pl.when(c == 1)
    def _():
        run(rel_hi)

    plsc.subcore_barrier()

    @pl.when(c == 0)
    def _():
        pltpu.sync_copy(acc.at[pl.ds(s * OPT, OPT)],
                        agg_lo.at[pl.ds(s * OPT, OPT)])

    @pl.when(c == 1)
    def _():
        pltpu.sync_copy(acc.at[pl.ds(s * OPT, OPT)],
                        agg_hi.at[pl.ds(s * OPT, OPT)])


@functools.lru_cache(maxsize=None)
def _sc_kernels():
    mesh = plsc.VectorSubcoreMesh(
        core_axis_name="c", subcore_axis_name="s",
        num_cores=NC, num_subcores=NS)
    params = pltpu.CompilerParams(use_tc_tiling_on_sc=False)
    embed = pl.kernel(
        _sc_embed_body,
        out_type=tuple(jax.ShapeDtypeStruct((NPAD, 16), _f32)
                       for _ in range(4)),
        mesh=mesh,
        scratch_types=[
            pltpu.VMEM((GT, CHUNK), _i32),
            pltpu.VMEM((CHUNK, 16), _f32),
            pltpu.SemaphoreType.DMA,
        ],
        compiler_params=params)
    edges = pl.kernel(
        _sc_edges_body,
        out_type=(pltpu.HBM((N, DH), _f32),
                  pltpu.HBM((N, DH), _f32)),
        mesh=mesh,
        scratch_types=[
            pltpu.VMEM((KB, CHUNK), _i32),
            pltpu.VMEM((KB, CHUNK), _i32),
            pltpu.VMEM((CHUNK, DH), _f32),
            pltpu.VMEM_SHARED((ACC_ROWS, DH), _f32),
            pltpu.SemaphoreType.DMA,
        ],
        compiler_params=params)
    return embed, edges


def _sc_embed(*args):
    return _sc_kernels()[0](*args)


def _sc_edges(*args):
    return _sc_kernels()[1](*args)


def _write_outs(o, lo_ref, hi_ref, self_ref):
    for r in range(R):
        lo_ref[r] = o[:, r * D:r * D + DH]
        hi_ref[r] = o[:, r * D + DH:(r + 1) * D]
    self_ref[...] = o[:, R * D:R * D + D]


def _tc_layer0_body(h0_ref, h1_ref, h2_ref, h3_ref, w_ref, lo_ref, hi_ref,
                    self_ref):
    w = w_ref[...]
    o = jnp.dot(h0_ref[...], w[0:16], preferred_element_type=_f32)
    for t, hr in enumerate((h1_ref, h2_ref, h3_ref), start=1):
        o += jnp.dot(hr[...], w[16 * t:16 * t + 16],
                     preferred_element_type=_f32)
    _write_outs(o, lo_ref, hi_ref, self_ref)


def _tc_layer_body(lo_in, hi_in, self_in, w_ref, lo_ref, hi_ref, self_ref):
    sp = self_in[...]
    hlo = jnp.maximum(lo_in[...] + sp[:, :DH], 0.0)
    hhi = jnp.maximum(hi_in[...] + sp[:, DH:], 0.0)
    w = w_ref[...]
    o = (jnp.dot(hlo, w[:DH], preferred_element_type=_f32)
         + jnp.dot(hhi, w[DH:], preferred_element_type=_f32))
    _write_outs(o, lo_ref, hi_ref, self_ref)


_LAYER_OUT = [
    jax.ShapeDtypeStruct((R, N, DH), _f32),
    jax.ShapeDtypeStruct((R, N, DH), _f32),
    jax.ShapeDtypeStruct((N, D), _f32),
]
_LAYER_OUT_SPECS = [
    pl.BlockSpec((R, BN, DH), lambda i: (0, i, 0)),
    pl.BlockSpec((R, BN, DH), lambda i: (0, i, 0)),
    pl.BlockSpec((BN, D), lambda i: (i, 0)),
]
_W_SPEC = pl.BlockSpec((D, R * D + D), lambda i: (0, 0))

_tc_layer0 = pl.pallas_call(
    _tc_layer0_body,
    grid=(NB,),
    in_specs=[pl.BlockSpec((BN, 16), lambda i: (i, 0)) for _ in range(4)]
    + [_W_SPEC],
    out_specs=_LAYER_OUT_SPECS,
    out_shape=_LAYER_OUT,
)

_tc_layer = pl.pallas_call(
    _tc_layer_body,
    grid=(NB,),
    in_specs=[
        pl.BlockSpec((BN, DH), lambda i: (i, 0)),
        pl.BlockSpec((BN, DH), lambda i: (i, 0)),
        pl.BlockSpec((BN, D), lambda i: (i, 0)),
        _W_SPEC,
    ],
    out_specs=_LAYER_OUT_SPECS,
    out_shape=_LAYER_OUT,
)


def _tc_head_body(lo_in, hi_in, self_in, pool_in, w1_ref, b1_ref, w2_ref,
                  b2_ref, out_ref, pooled):
    i = pl.program_id(0)

    @pl.when(i == 0)
    def _():
        pooled[...] = jnp.zeros_like(pooled)

    sp = self_in[...]
    hlo = jnp.maximum(lo_in[...] + sp[:, :DH], 0.0)
    hhi = jnp.maximum(hi_in[...] + sp[:, DH:], 0.0)
    pm = pool_in[...]
    dn = (((0,), (0,)), ((), ()))
    pooled[:, :DH] += lax.dot_general(pm, hlo, dn,
                                      preferred_element_type=_f32)
    pooled[:, DH:] += lax.dot_general(pm, hhi, dn,
                                      preferred_element_type=_f32)

    @pl.when(i == NB - 1)
    def _():
        p = pooled[...]
        hid = jnp.maximum(
            jnp.dot(p, w1_ref[...], preferred_element_type=_f32)
            + b1_ref[...], 0.0)
        out_ref[...] = (jnp.dot(hid, w2_ref[...],
                                preferred_element_type=_f32) + b2_ref[...])


_tc_head = pl.pallas_call(
    _tc_head_body,
    grid=(NB,),
    in_specs=[
        pl.BlockSpec((BN, DH), lambda i: (i, 0)),
        pl.BlockSpec((BN, DH), lambda i: (i, 0)),
        pl.BlockSpec((BN, D), lambda i: (i, 0)),
        pl.BlockSpec((BN, BATCH), lambda i: (i, 0)),
        pl.BlockSpec((D, D), lambda i: (0, 0)),
        pl.BlockSpec((1, D), lambda i: (0, 0)),
        pl.BlockSpec((D, 128), lambda i: (0, 0)),
        pl.BlockSpec((1, 128), lambda i: (0, 0)),
    ],
    out_specs=pl.BlockSpec((BATCH, 128), lambda i: (0, 0)),
    out_shape=jax.ShapeDtypeStruct((BATCH, 128), _f32),
    scratch_shapes=[pltpu.VMEM((BATCH, D), _f32)],
)


def kernel(feats, edge_index, etype, emb0, emb1, emb2, emb3, W_rel, W_self,
           fcn_W1, fcn_b1, fcn_W2, fcn_b2):
    feats = feats.astype(_i32)
    src = edge_index[0].astype(_i32)
    dst = edge_index[1].astype(_i32)
    etype = etype.astype(_i32)

    flat = etype * N + src
    flat_p = jnp.pad(flat, (0, EPAD - E)).reshape(NS, G, CHUNK)
    dst_p = jnp.pad(dst, (0, EPAD - E),
                    constant_values=N).reshape(NS, G, CHUNK)

    fidx = jnp.pad(feats.T, ((0, 0), (0, NPAD - N))).reshape(4, NW, GT, CHUNK)

    w_cat = jnp.concatenate(
        [W_rel.transpose(0, 2, 1, 3).reshape(NL, D, R * D), W_self], axis=-1)

    zeros_acc = jnp.zeros((ZROWS, DH), _f32)
    pool_mat = jnp.where(
        (jnp.arange(N, dtype=_i32)[:, None] // NPG)
        == jnp.arange(BATCH, dtype=_i32)[None, :],
        jnp.float32(1.0 / NPG), jnp.float32(0.0))
    w2p = jnp.pad(fcn_W2, ((0, 0), (0, 128 - 1)))
    b2p = jnp.pad(fcn_b2, (0, 128 - 1)).reshape(1, 128)
    b1r = fcn_b1.reshape(1, D)

    h0, h1, h2, h3 = _sc_embed(fidx, emb0, emb1, emb2, emb3)

    rel_lo, rel_hi, selfo = _tc_layer0(h0[:N], h1[:N], h2[:N], h3[:N],
                                       w_cat[0])
    for l in range(NL):
        agg_lo, agg_hi = _sc_edges(
            rel_lo.reshape(R * N, DH), rel_hi.reshape(R * N, DH),
            flat_p, dst_p, zeros_acc)
        if l < NL - 1:
            rel_lo, rel_hi, selfo = _tc_layer(agg_lo, agg_hi, selfo,
                                              w_cat[l + 1])

    out = _tc_head(agg_lo, agg_hi, selfo, pool_mat, fcn_W1, b1r, w2p, b2p)
    return out[:, :1]


# speedup vs baseline: 2.3572x; 1.0923x over previous
import functools

import jax
import jax.numpy as jnp
from jax import lax
from jax.experimental import pallas as pl
from jax.experimental.pallas import tpu as pltpu
from jax.experimental.pallas import tpu_sc as plsc

N = 50000
E = 800000
R = 8
D = 64
DH = 32
NL = 4
BATCH = 16
NPG = 3125

NC = 2
NS = 16
CHUNK = 128

G = (E + NS * CHUNK - 1) // (NS * CHUNK)
EPT = G * CHUNK
EPAD = NS * EPT
KB = 23
NBLK = G // KB

NW = NC * NS
GT = (N + NW * CHUNK - 1) // (NW * CHUNK)
NPT = GT * CHUNK
NPAD = NW * NPT

ACC_ROWS = NS * ((N + NS - 1) // NS + 1)
ZROWS = ACC_ROWS // NS
OPT = N // NS

BN = 2000
NB = N // BN

_f32 = jnp.float32
_i32 = jnp.int32


def _sc_embed_body(fidx, e0, e1, e2, e3, h0, h1, h2, h3, idx_v, rows_v, sem):
    c = lax.axis_index("c")
    s = lax.axis_index("s")
    w = s * NC + c
    for t, (et, ht) in enumerate(zip((e0, e1, e2, e3), (h0, h1, h2, h3))):
        pltpu.sync_copy(fidx.at[t, w], idx_v)
        for g in range(GT):
            pltpu.async_copy(et.at[idx_v.at[g]], rows_v, sem).wait()
            pltpu.sync_copy(
                rows_v, ht.at[pl.ds(w * NPT + g * CHUNK, CHUNK)])


RING = 4
LOOKAHEAD = 2


def _sc_edges_body(rel_all, flat_i, dst_i, zeros_hbm, agg_lo, agg_hi,
                   flat_v, dst_v, rows_v, acc, sem_g, sem_s):
    c = lax.axis_index("c")
    s = lax.axis_index("s")
    pltpu.sync_copy(zeros_hbm, acc.at[pl.ds(s * ZROWS, ZROWS)])
    plsc.subcore_barrier()

    def gather(j):
        pltpu.async_copy(rel_all.at[flat_v.at[j]], rows_v.at[j % RING], sem_g)

    def wait_gather(j):
        pltpu.make_async_copy(rel_all.at[flat_v.at[j]], rows_v.at[j % RING],
                              sem_g).wait()

    def scatter(j):
        pltpu.async_copy(rows_v.at[j % RING], acc.at[dst_v.at[j]], sem_s,
                         add=True)

    def wait_scatter(j):
        pltpu.make_async_copy(rows_v.at[j % RING], acc.at[dst_v.at[j]],
                              sem_s).wait()

    def outer(b, carry):
        pltpu.sync_copy(flat_i.at[c, s, pl.ds(b * KB, KB)], flat_v)
        pltpu.sync_copy(dst_i.at[s, pl.ds(b * KB, KB)], dst_v)
        for j in range(LOOKAHEAD):
            gather(j)
        for j in range(KB):
            wait_gather(j)
            nxt = j + LOOKAHEAD
            if nxt < KB:
                if j >= LOOKAHEAD:
                    wait_scatter(j - LOOKAHEAD)
                gather(nxt)
            scatter(j)
        for j in range(KB - 2 * LOOKAHEAD, KB):
            wait_scatter(j)
        return carry

    lax.fori_loop(0, NBLK, outer, 0)
    plsc.subcore_barrier()

    @pl.when(c == 0)
    def _():
        pltpu.sync_copy(acc.at[pl.ds(s * OPT, OPT)],
                        agg_lo.at[pl.ds(s * OPT, OPT)])

    @pl.when(c == 1)
    def _():
        pltpu.sync_copy(acc.at[pl.ds(s * OPT, OPT)],
                        agg_hi.at[pl.ds(s * OPT, OPT)])


@functools.lru_cache(maxsize=None)
def _sc_kernels():
    mesh = plsc.VectorSubcoreMesh(
        core_axis_name="c", subcore_axis_name="s",
        num_cores=NC, num_subcores=NS)
    params = pltpu.CompilerParams(use_tc_tiling_on_sc=False)
    embed = pl.kernel(
        _sc_embed_body,
        out_type=tuple(jax.ShapeDtypeStruct((NPAD, 16), _f32)
                       for _ in range(4)),
        mesh=mesh,
        scratch_types=[
            pltpu.VMEM((GT, CHUNK), _i32),
            pltpu.VMEM((CHUNK, 16), _f32),
            pltpu.SemaphoreType.DMA,
        ],
        compiler_params=params)
    edges = pl.kernel(
        _sc_edges_body,
        out_type=(pltpu.HBM((N, DH), _f32),
                  pltpu.HBM((N, DH), _f32)),
        mesh=mesh,
        scratch_types=[
            pltpu.VMEM((KB, CHUNK), _i32),
            pltpu.VMEM((KB, CHUNK), _i32),
            pltpu.VMEM((RING, CHUNK, DH), _f32),
            pltpu.VMEM_SHARED((ACC_ROWS, DH), _f32),
            pltpu.SemaphoreType.DMA,
            pltpu.SemaphoreType.DMA,
        ],
        compiler_params=params)
    return embed, edges


def _sc_embed(*args):
    return _sc_kernels()[0](*args)


def _sc_edges(*args):
    return _sc_kernels()[1](*args)


def _write_outs(o, rel_ref, self_ref):
    for r in range(R):
        rel_ref[r] = o[:, r * D:r * D + DH]
        rel_ref[R + r] = o[:, r * D + DH:(r + 1) * D]
    self_ref[...] = o[:, R * D:R * D + D]


def _tc_layer0_body(h0_ref, h1_ref, h2_ref, h3_ref, w_ref, rel_ref, self_ref):
    w = w_ref[...]
    o = jnp.dot(h0_ref[...], w[0:16], preferred_element_type=_f32, precision=lax.Precision.HIGHEST)
    for t, hr in enumerate((h1_ref, h2_ref, h3_ref), start=1):
        o += jnp.dot(hr[...], w[16 * t:16 * t + 16],
                     preferred_element_type=_f32, precision=lax.Precision.HIGHEST)
    _write_outs(o, rel_ref, self_ref)


def _tc_layer_body(lo_in, hi_in, self_in, w_ref, rel_ref, self_ref):
    sp = self_in[...]
    hlo = jnp.maximum(lo_in[...] + sp[:, :DH], 0.0)
    hhi = jnp.maximum(hi_in[...] + sp[:, DH:], 0.0)
    w = w_ref[...]
    o = (jnp.dot(hlo, w[:DH], preferred_element_type=_f32, precision=lax.Precision.HIGHEST)
         + jnp.dot(hhi, w[DH:], preferred_element_type=_f32, precision=lax.Precision.HIGHEST))
    _write_outs(o, rel_ref, self_ref)


_LAYER_OUT = [
    jax.ShapeDtypeStruct((2 * R, N, DH), _f32),
    jax.ShapeDtypeStruct((N, D), _f32),
]
_LAYER_OUT_SPECS = [
    pl.BlockSpec((2 * R, BN, DH), lambda i: (0, i, 0)),
    pl.BlockSpec((BN, D), lambda i: (i, 0)),
]
_W_SPEC = pl.BlockSpec((D, R * D + D), lambda i: (0, 0))

_tc_layer0 = pl.pallas_call(
    _tc_layer0_body,
    grid=(NB,),
    in_specs=[pl.BlockSpec((BN, 16), lambda i: (i, 0)) for _ in range(4)]
    + [_W_SPEC],
    out_specs=_LAYER_OUT_SPECS,
    out_shape=_LAYER_OUT,
)

_tc_layer = pl.pallas_call(
    _tc_layer_body,
    grid=(NB,),
    in_specs=[
        pl.BlockSpec((BN, DH), lambda i: (i, 0)),
        pl.BlockSpec((BN, DH), lambda i: (i, 0)),
        pl.BlockSpec((BN, D), lambda i: (i, 0)),
        _W_SPEC,
    ],
    out_specs=_LAYER_OUT_SPECS,
    out_shape=_LAYER_OUT,
)


def _tc_head_body(lo_in, hi_in, self_in, pool_in, w1_ref, b1_ref, w2_ref,
                  b2_ref, out_ref, pooled):
    i = pl.program_id(0)

    @pl.when(i == 0)
    def _():
        pooled[...] = jnp.zeros_like(pooled)

    sp = self_in[...]
    hlo = jnp.maximum(lo_in[...] + sp[:, :DH], 0.0)
    hhi = jnp.maximum(hi_in[...] + sp[:, DH:], 0.0)
    pm = pool_in[...]
    dn = (((0,), (0,)), ((), ()))
    pooled[:, :DH] += lax.dot_general(pm, hlo, dn,
                                      preferred_element_type=_f32, precision=lax.Precision.HIGHEST)
    pooled[:, DH:] += lax.dot_general(pm, hhi, dn,
                                      preferred_element_type=_f32, precision=lax.Precision.HIGHEST)

    @pl.when(i == NB - 1)
    def _():
        p = pooled[...]
        hid = jnp.maximum(
            jnp.dot(p, w1_ref[...], preferred_element_type=_f32, precision=lax.Precision.HIGHEST)
            + b1_ref[...], 0.0)
        out_ref[...] = (jnp.dot(hid, w2_ref[...],
                                preferred_element_type=_f32, precision=lax.Precision.HIGHEST) + b2_ref[...])


_tc_head = pl.pallas_call(
    _tc_head_body,
    grid=(NB,),
    in_specs=[
        pl.BlockSpec((BN, DH), lambda i: (i, 0)),
        pl.BlockSpec((BN, DH), lambda i: (i, 0)),
        pl.BlockSpec((BN, D), lambda i: (i, 0)),
        pl.BlockSpec((BN, BATCH), lambda i: (i, 0)),
        pl.BlockSpec((D, D), lambda i: (0, 0)),
        pl.BlockSpec((1, D), lambda i: (0, 0)),
        pl.BlockSpec((D, 128), lambda i: (0, 0)),
        pl.BlockSpec((1, 128), lambda i: (0, 0)),
    ],
    out_specs=pl.BlockSpec((BATCH, 128), lambda i: (0, 0)),
    out_shape=jax.ShapeDtypeStruct((BATCH, 128), _f32),
    scratch_shapes=[pltpu.VMEM((BATCH, D), _f32)],
)


def kernel(feats, edge_index, etype, emb0, emb1, emb2, emb3, W_rel, W_self,
           fcn_W1, fcn_b1, fcn_W2, fcn_b2):
    feats = feats.astype(_i32)
    src = edge_index[0].astype(_i32)
    dst = edge_index[1].astype(_i32)
    etype = etype.astype(_i32)

    flat = etype * N + src
    flat_p = jnp.pad(flat, (0, EPAD - E)).reshape(NS, G, CHUNK)
    flat2 = jnp.stack([flat_p, flat_p + R * N])
    dst_p = jnp.pad(dst, (0, EPAD - E),
                    constant_values=N).reshape(NS, G, CHUNK)

    fidx = jnp.pad(feats.T, ((0, 0), (0, NPAD - N))).reshape(4, NW, GT, CHUNK)

    w_cat = jnp.concatenate(
        [W_rel.transpose(0, 2, 1, 3).reshape(NL, D, R * D), W_self], axis=-1)

    zeros_acc = jnp.zeros((ZROWS, DH), _f32)
    pool_mat = jnp.where(
        (jnp.arange(N, dtype=_i32)[:, None] // NPG)
        == jnp.arange(BATCH, dtype=_i32)[None, :],
        jnp.float32(1.0 / NPG), jnp.float32(0.0))
    w2p = jnp.pad(fcn_W2, ((0, 0), (0, 128 - 1)))
    b2p = jnp.pad(fcn_b2, (0, 128 - 1)).reshape(1, 128)
    b1r = fcn_b1.reshape(1, D)

    h0, h1, h2, h3 = _sc_embed(fidx, emb0, emb1, emb2, emb3)

    rel, selfo = _tc_layer0(h0[:N], h1[:N], h2[:N], h3[:N], w_cat[0])
    for l in range(NL):
        agg_lo, agg_hi = _sc_edges(
            rel.reshape(2 * R * N, DH), flat2, dst_p, zeros_acc)
        if l < NL - 1:
            rel, selfo = _tc_layer(agg_lo, agg_hi, selfo, w_cat[l + 1])

    out = _tc_head(agg_lo, agg_hi, selfo, pool_mat, fcn_W1, b1r, w2p, b2p)
    return out[:, :1]


# speedup vs baseline: 3.0998x; 1.3150x over previous
import functools

import jax
import jax.numpy as jnp
from jax import lax
from jax.experimental import pallas as pl
from jax.experimental.pallas import tpu as pltpu
from jax.experimental.pallas import tpu_sc as plsc

N = 50000
E = 800000
R = 8
D = 64
DH = 32
NL = 4
BATCH = 16
NPG = 3125

NC = 2
NS = 16
CHUNK = 128

G = (E + NS * CHUNK - 1) // (NS * CHUNK)
EPT = G * CHUNK
EPAD = NS * EPT
KB = 23
NBLK = G // KB

NW = NC * NS
GT = (N + NW * CHUNK - 1) // (NW * CHUNK)
NPT = GT * CHUNK
NPAD = NW * NPT

ACC_ROWS = NS * ((N + NS - 1) // NS + 1)
ZROWS = ACC_ROWS // NS
OPT = N // NS

BN = 2000
NB = N // BN

_f32 = jnp.float32
_i32 = jnp.int32


def _sc_embed_body(fidx, e0, e1, e2, e3, h0, h1, h2, h3, idx_v, rows_v, sem):
    c = lax.axis_index("c")
    s = lax.axis_index("s")
    w = s * NC + c
    for t, (et, ht) in enumerate(zip((e0, e1, e2, e3), (h0, h1, h2, h3))):
        pltpu.sync_copy(fidx.at[t, w], idx_v)
        for g in range(GT):
            pltpu.async_copy(et.at[idx_v.at[g]], rows_v, sem).wait()
            pltpu.sync_copy(
                rows_v, ht.at[pl.ds(w * NPT + g * CHUNK, CHUNK)])


RING = 4
LOOKAHEAD = 2


def _sc_edges_body(rel_all, flat_i, dst_i, zeros_hbm, agg_lo, agg_hi,
                   flat_v, dst_v, rows_v, acc, sem_g, sem_s):
    c = lax.axis_index("c")
    s = lax.axis_index("s")
    pltpu.sync_copy(zeros_hbm, acc.at[pl.ds(s * ZROWS, ZROWS)])
    plsc.subcore_barrier()

    def gather(j):
        pltpu.async_copy(rel_all.at[flat_v.at[j]], rows_v.at[j % RING], sem_g)

    def wait_gather(j):
        pltpu.make_async_copy(rel_all.at[flat_v.at[j]], rows_v.at[j % RING],
                              sem_g).wait()

    def scatter(j):
        pltpu.async_copy(rows_v.at[j % RING], acc.at[dst_v.at[j]], sem_s,
                         add=True)

    def wait_scatter(j):
        pltpu.make_async_copy(rows_v.at[j % RING], acc.at[dst_v.at[j]],
                              sem_s).wait()

    def outer(b, carry):
        pltpu.sync_copy(flat_i.at[c, s, pl.ds(b * KB, KB)], flat_v)
        pltpu.sync_copy(dst_i.at[s, pl.ds(b * KB, KB)], dst_v)
        for j in range(LOOKAHEAD):
            gather(j)
        for j in range(KB):
            wait_gather(j)
            nxt = j + LOOKAHEAD
            if nxt < KB:
                if j >= LOOKAHEAD:
                    wait_scatter(j - LOOKAHEAD)
                gather(nxt)
            scatter(j)
        for j in range(KB - 2 * LOOKAHEAD, KB):
            wait_scatter(j)
        return carry

    lax.fori_loop(0, NBLK, outer, 0)
    plsc.subcore_barrier()

    @pl.when(c == 0)
    def _():
        pltpu.sync_copy(acc.at[pl.ds(s * OPT, OPT)],
                        agg_lo.at[pl.ds(s * OPT, OPT)])

    @pl.when(c == 1)
    def _():
        pltpu.sync_copy(acc.at[pl.ds(s * OPT, OPT)],
                        agg_hi.at[pl.ds(s * OPT, OPT)])


@functools.lru_cache(maxsize=None)
def _sc_kernels():
    mesh = plsc.VectorSubcoreMesh(
        core_axis_name="c", subcore_axis_name="s",
        num_cores=NC, num_subcores=NS)
    params = pltpu.CompilerParams(use_tc_tiling_on_sc=False)
    embed = pl.kernel(
        _sc_embed_body,
        out_type=tuple(jax.ShapeDtypeStruct((NPAD, 16), _f32)
                       for _ in range(4)),
        mesh=mesh,
        scratch_types=[
            pltpu.VMEM((GT, CHUNK), _i32),
            pltpu.VMEM((CHUNK, 16), _f32),
            pltpu.SemaphoreType.DMA,
        ],
        compiler_params=params)
    edges = pl.kernel(
        _sc_edges_body,
        out_type=(pltpu.HBM((N, DH), _f32),
                  pltpu.HBM((N, DH), _f32)),
        mesh=mesh,
        scratch_types=[
            pltpu.VMEM((KB, CHUNK), _i32),
            pltpu.VMEM((KB, CHUNK), _i32),
            pltpu.VMEM((RING, CHUNK, DH), _f32),
            pltpu.VMEM_SHARED((ACC_ROWS, DH), _f32),
            pltpu.SemaphoreType.DMA,
            pltpu.SemaphoreType.DMA,
        ],
        compiler_params=params)
    return embed, edges


def _sc_embed(*args):
    return _sc_kernels()[0](*args)


def _sc_edges(*args):
    return _sc_kernels()[1](*args)


def _write_outs(o, rel_ref, self_ref):
    rel_ref[...] = o[:, :R * D]
    self_ref[...] = o[:, R * D:]


def _tc_layer0_body(h0_ref, h1_ref, h2_ref, h3_ref, w_ref, rel_ref, self_ref):
    w = w_ref[...]
    o = jnp.dot(h0_ref[...], w[0:16], preferred_element_type=_f32, precision=lax.Precision.HIGHEST)
    for t, hr in enumerate((h1_ref, h2_ref, h3_ref), start=1):
        o += jnp.dot(hr[...], w[16 * t:16 * t + 16],
                     preferred_element_type=_f32, precision=lax.Precision.HIGHEST)
    _write_outs(o, rel_ref, self_ref)


def _tc_layer_body(lo_in, hi_in, self_in, w_ref, rel_ref, self_ref):
    sp = self_in[...]
    hlo = jnp.maximum(lo_in[...] + sp[:, :DH], 0.0)
    hhi = jnp.maximum(hi_in[...] + sp[:, DH:], 0.0)
    w = w_ref[...]
    o = (jnp.dot(hlo, w[:DH], preferred_element_type=_f32, precision=lax.Precision.HIGHEST)
         + jnp.dot(hhi, w[DH:], preferred_element_type=_f32, precision=lax.Precision.HIGHEST))
    _write_outs(o, rel_ref, self_ref)


_LAYER_OUT = [
    jax.ShapeDtypeStruct((N, R * D), _f32),
    jax.ShapeDtypeStruct((N, D), _f32),
]
_LAYER_OUT_SPECS = [
    pl.BlockSpec((BN, R * D), lambda i: (i, 0)),
    pl.BlockSpec((BN, D), lambda i: (i, 0)),
]
_W_SPEC = pl.BlockSpec((D, R * D + D), lambda i: (0, 0))

_tc_layer0 = pl.pallas_call(
    _tc_layer0_body,
    grid=(NB,),
    in_specs=[pl.BlockSpec((BN, 16), lambda i: (i, 0)) for _ in range(4)]
    + [_W_SPEC],
    out_specs=_LAYER_OUT_SPECS,
    out_shape=_LAYER_OUT,
)

_tc_layer = pl.pallas_call(
    _tc_layer_body,
    grid=(NB,),
    in_specs=[
        pl.BlockSpec((BN, DH), lambda i: (i, 0)),
        pl.BlockSpec((BN, DH), lambda i: (i, 0)),
        pl.BlockSpec((BN, D), lambda i: (i, 0)),
        _W_SPEC,
    ],
    out_specs=_LAYER_OUT_SPECS,
    out_shape=_LAYER_OUT,
)


def _tc_head_body(lo_in, hi_in, self_in, pool_in, w1_ref, b1_ref, w2_ref,
                  b2_ref, out_ref, pooled):
    i = pl.program_id(0)

    @pl.when(i == 0)
    def _():
        pooled[...] = jnp.zeros_like(pooled)

    sp = self_in[...]
    hlo = jnp.maximum(lo_in[...] + sp[:, :DH], 0.0)
    hhi = jnp.maximum(hi_in[...] + sp[:, DH:], 0.0)
    pm = pool_in[...]
    dn = (((0,), (0,)), ((), ()))
    pooled[:, :DH] += lax.dot_general(pm, hlo, dn,
                                      preferred_element_type=_f32, precision=lax.Precision.HIGHEST)
    pooled[:, DH:] += lax.dot_general(pm, hhi, dn,
                                      preferred_element_type=_f32, precision=lax.Precision.HIGHEST)

    @pl.when(i == NB - 1)
    def _():
        p = pooled[...]
        hid = jnp.maximum(
            jnp.dot(p, w1_ref[...], preferred_element_type=_f32, precision=lax.Precision.HIGHEST)
            + b1_ref[...], 0.0)
        out_ref[...] = (jnp.dot(hid, w2_ref[...],
                                preferred_element_type=_f32, precision=lax.Precision.HIGHEST) + b2_ref[...])


_tc_head = pl.pallas_call(
    _tc_head_body,
    grid=(NB,),
    in_specs=[
        pl.BlockSpec((BN, DH), lambda i: (i, 0)),
        pl.BlockSpec((BN, DH), lambda i: (i, 0)),
        pl.BlockSpec((BN, D), lambda i: (i, 0)),
        pl.BlockSpec((BN, BATCH), lambda i: (i, 0)),
        pl.BlockSpec((D, D), lambda i: (0, 0)),
        pl.BlockSpec((1, D), lambda i: (0, 0)),
        pl.BlockSpec((D, 128), lambda i: (0, 0)),
        pl.BlockSpec((1, 128), lambda i: (0, 0)),
    ],
    out_specs=pl.BlockSpec((BATCH, 128), lambda i: (0, 0)),
    out_shape=jax.ShapeDtypeStruct((BATCH, 128), _f32),
    scratch_shapes=[pltpu.VMEM((BATCH, D), _f32)],
)


def kernel(feats, edge_index, etype, emb0, emb1, emb2, emb3, W_rel, W_self,
           fcn_W1, fcn_b1, fcn_W2, fcn_b2):
    feats = feats.astype(_i32)
    src = edge_index[0].astype(_i32)
    dst = edge_index[1].astype(_i32)
    etype = etype.astype(_i32)

    flat = src * (2 * R) + etype * 2
    flat_p = jnp.pad(flat, (0, EPAD - E)).reshape(NS, G, CHUNK)
    flat2 = jnp.stack([flat_p, flat_p + 1])
    dst_p = jnp.pad(dst, (0, EPAD - E),
                    constant_values=N).reshape(NS, G, CHUNK)

    fidx = jnp.pad(feats.T, ((0, 0), (0, NPAD - N))).reshape(4, NW, GT, CHUNK)

    w_cat = jnp.concatenate(
        [W_rel.transpose(0, 2, 1, 3).reshape(NL, D, R * D), W_self], axis=-1)

    zeros_acc = jnp.zeros((ZROWS, DH), _f32)
    pool_mat = jnp.where(
        (jnp.arange(N, dtype=_i32)[:, None] // NPG)
        == jnp.arange(BATCH, dtype=_i32)[None, :],
        jnp.float32(1.0 / NPG), jnp.float32(0.0))
    w2p = jnp.pad(fcn_W2, ((0, 0), (0, 128 - 1)))
    b2p = jnp.pad(fcn_b2, (0, 128 - 1)).reshape(1, 128)
    b1r = fcn_b1.reshape(1, D)

    h0, h1, h2, h3 = _sc_embed(fidx, emb0, emb1, emb2, emb3)

    rel, selfo = _tc_layer0(h0[:N], h1[:N], h2[:N], h3[:N], w_cat[0])
    for l in range(NL):
        agg_lo, agg_hi = _sc_edges(
            rel.reshape(2 * R * N, DH), flat2, dst_p, zeros_acc)
        if l < NL - 1:
            rel, selfo = _tc_layer(agg_lo, agg_hi, selfo, w_cat[l + 1])

    out = _tc_head(agg_lo, agg_hi, selfo, pool_mat, fcn_W1, b1r, w2p, b2p)
    return out[:, :1]


# speedup vs baseline: 4.1784x; 1.3480x over previous
import functools

import jax
import jax.numpy as jnp
from jax import lax
from jax.experimental import pallas as pl
from jax.experimental.pallas import tpu as pltpu
from jax.experimental.pallas import tpu_sc as plsc

N = 50000
E = 800000
R = 8
D = 64
DH = 32
NL = 4
BATCH = 16
NPG = 3125

NC = 2
NS = 16
CHUNK = 128

G = (E + NS * CHUNK - 1) // (NS * CHUNK)
EPT = G * CHUNK
EPAD = NS * EPT
KB = 23
NBLK = G // KB

NW = NC * NS
GT = (N + NW * CHUNK - 1) // (NW * CHUNK)
NPT = GT * CHUNK
NPAD = NW * NPT

ACC_ROWS = NS * ((N + NS - 1) // NS + 1)
ZROWS = ACC_ROWS // NS
OPT = N // NS

BN = 2000
NB = N // BN

_f32 = jnp.float32
_i32 = jnp.int32


def _sc_embed_body(fidx, e0, e1, e2, e3, h0, h1, h2, h3, idx_v, rows_v, sem):
    c = lax.axis_index("c")
    s = lax.axis_index("s")
    w = s * NC + c
    for t, (et, ht) in enumerate(zip((e0, e1, e2, e3), (h0, h1, h2, h3))):
        pltpu.sync_copy(fidx.at[t, w], idx_v)
        for g in range(GT):
            pltpu.async_copy(et.at[idx_v.at[g]], rows_v, sem).wait()
            pltpu.sync_copy(
                rows_v, ht.at[pl.ds(w * NPT + g * CHUNK, CHUNK)])


RING = 4
LOOKAHEAD = 2


def _sc_edges_body(rel_all, flat_i, dst_i, zeros_hbm, agg_lo, agg_hi,
                   flat_v, dst_v, rows_v, acc, *sems):
    c = lax.axis_index("c")
    s = lax.axis_index("s")
    sem_g, sem_s = sems[:RING], sems[RING:]
    pltpu.sync_copy(zeros_hbm, acc.at[pl.ds(s * ZROWS, ZROWS)])
    plsc.subcore_barrier()

    def gather(j):
        pltpu.async_copy(rel_all.at[flat_v.at[j]], rows_v.at[j % RING],
                         sem_g[j % RING])

    def wait_gather(j):
        pltpu.make_async_copy(rel_all.at[flat_v.at[j]], rows_v.at[j % RING],
                              sem_g[j % RING]).wait()

    def scatter(j):
        pltpu.async_copy(rows_v.at[j % RING], acc.at[dst_v.at[j]],
                         sem_s[j % RING], add=True)

    def wait_scatter(j):
        pltpu.make_async_copy(rows_v.at[j % RING], acc.at[dst_v.at[j]],
                              sem_s[j % RING]).wait()

    def outer(b, carry):
        pltpu.sync_copy(flat_i.at[c, s, pl.ds(b * KB, KB)], flat_v)
        pltpu.sync_copy(dst_i.at[s, pl.ds(b * KB, KB)], dst_v)
        for j in range(LOOKAHEAD):
            gather(j)
        for j in range(KB):
            wait_gather(j)
            nxt = j + LOOKAHEAD
            if nxt < KB:
                if j >= LOOKAHEAD:
                    wait_scatter(j - LOOKAHEAD)
                gather(nxt)
            scatter(j)
        for j in range(KB - 2 * LOOKAHEAD, KB):
            wait_scatter(j)
        return carry

    lax.fori_loop(0, NBLK, outer, 0)
    plsc.subcore_barrier()

    @pl.when(c == 0)
    def _():
        pltpu.sync_copy(acc.at[pl.ds(s * OPT, OPT)],
                        agg_lo.at[pl.ds(s * OPT, OPT)])

    @pl.when(c == 1)
    def _():
        pltpu.sync_copy(acc.at[pl.ds(s * OPT, OPT)],
                        agg_hi.at[pl.ds(s * OPT, OPT)])


@functools.lru_cache(maxsize=None)
def _sc_kernels():
    mesh = plsc.VectorSubcoreMesh(
        core_axis_name="c", subcore_axis_name="s",
        num_cores=NC, num_subcores=NS)
    params = pltpu.CompilerParams(use_tc_tiling_on_sc=False)
    embed = pl.kernel(
        _sc_embed_body,
        out_type=tuple(jax.ShapeDtypeStruct((NPAD, 16), _f32)
                       for _ in range(4)),
        mesh=mesh,
        scratch_types=[
            pltpu.VMEM((GT, CHUNK), _i32),
            pltpu.VMEM((CHUNK, 16), _f32),
            pltpu.SemaphoreType.DMA,
        ],
        compiler_params=params)
    edges = pl.kernel(
        _sc_edges_body,
        out_type=(pltpu.HBM((N, DH), _f32),
                  pltpu.HBM((N, DH), _f32)),
        mesh=mesh,
        scratch_types=[
            pltpu.VMEM((KB, CHUNK), _i32),
            pltpu.VMEM((KB, CHUNK), _i32),
            pltpu.VMEM((RING, CHUNK, DH), _f32),
            pltpu.VMEM_SHARED((ACC_ROWS, DH), _f32),
        ] + [pltpu.SemaphoreType.DMA] * (2 * RING),
        compiler_params=params)
    return embed, edges


def _sc_embed(*args):
    return _sc_kernels()[0](*args)


def _sc_edges(*args):
    return _sc_kernels()[1](*args)


def _write_outs(o, rel_ref, self_ref):
    rel_ref[...] = o[:, :R * D]
    self_ref[...] = o[:, R * D:]


def _tc_layer0_body(h0_ref, h1_ref, h2_ref, h3_ref, w_ref, rel_ref, self_ref):
    w = w_ref[...]
    o = jnp.dot(h0_ref[...], w[0:16], preferred_element_type=_f32)
    for t, hr in enumerate((h1_ref, h2_ref, h3_ref), start=1):
        o += jnp.dot(hr[...], w[16 * t:16 * t + 16],
                     preferred_element_type=_f32)
    _write_outs(o, rel_ref, self_ref)


def _tc_layer_body(lo_in, hi_in, self_in, w_ref, rel_ref, self_ref):
    sp = self_in[...]
    hlo = jnp.maximum(lo_in[...] + sp[:, :DH], 0.0)
    hhi = jnp.maximum(hi_in[...] + sp[:, DH:], 0.0)
    w = w_ref[...]
    o = (jnp.dot(hlo, w[:DH], preferred_element_type=_f32)
         + jnp.dot(hhi, w[DH:], preferred_element_type=_f32))
    _write_outs(o, rel_ref, self_ref)


_LAYER_OUT = [
    jax.ShapeDtypeStruct((N, R * D), _f32),
    jax.ShapeDtypeStruct((N, D), _f32),
]
_LAYER_OUT_SPECS = [
    pl.BlockSpec((BN, R * D), lambda i: (i, 0)),
    pl.BlockSpec((BN, D), lambda i: (i, 0)),
]
_W_SPEC = pl.BlockSpec((D, R * D + D), lambda i: (0, 0))

_tc_layer0 = pl.pallas_call(
    _tc_layer0_body,
    grid=(NB,),
    in_specs=[pl.BlockSpec((BN, 16), lambda i: (i, 0)) for _ in range(4)]
    + [_W_SPEC],
    out_specs=_LAYER_OUT_SPECS,
    out_shape=_LAYER_OUT,
)

_tc_layer = pl.pallas_call(
    _tc_layer_body,
    grid=(NB,),
    in_specs=[
        pl.BlockSpec((BN, DH), lambda i: (i, 0)),
        pl.BlockSpec((BN, DH), lambda i: (i, 0)),
        pl.BlockSpec((BN, D), lambda i: (i, 0)),
        _W_SPEC,
    ],
    out_specs=_LAYER_OUT_SPECS,
    out_shape=_LAYER_OUT,
)


def _tc_head_body(lo_in, hi_in, self_in, pool_in, w1_ref, b1_ref, w2_ref,
                  b2_ref, out_ref, pooled):
    i = pl.program_id(0)

    @pl.when(i == 0)
    def _():
        pooled[...] = jnp.zeros_like(pooled)

    sp = self_in[...]
    hlo = jnp.maximum(lo_in[...] + sp[:, :DH], 0.0)
    hhi = jnp.maximum(hi_in[...] + sp[:, DH:], 0.0)
    pm = pool_in[...]
    dn = (((0,), (0,)), ((), ()))
    pooled[:, :DH] += lax.dot_general(pm, hlo, dn,
                                      preferred_element_type=_f32,
                                      precision=lax.Precision.HIGHEST)
    pooled[:, DH:] += lax.dot_general(pm, hhi, dn,
                                      preferred_element_type=_f32,
                                      precision=lax.Precision.HIGHEST)

    @pl.when(i == NB - 1)
    def _():
        p = pooled[...]
        hid = jnp.maximum(
            jnp.dot(p, w1_ref[...], preferred_element_type=_f32)
            + b1_ref[...], 0.0)
        out_ref[...] = (jnp.dot(hid, w2_ref[...],
                                preferred_element_type=_f32) + b2_ref[...])


_tc_head = pl.pallas_call(
    _tc_head_body,
    grid=(NB,),
    in_specs=[
        pl.BlockSpec((BN, DH), lambda i: (i, 0)),
        pl.BlockSpec((BN, DH), lambda i: (i, 0)),
        pl.BlockSpec((BN, D), lambda i: (i, 0)),
        pl.BlockSpec((BN, BATCH), lambda i: (i, 0)),
        pl.BlockSpec((D, D), lambda i: (0, 0)),
        pl.BlockSpec((1, D), lambda i: (0, 0)),
        pl.BlockSpec((D, 128), lambda i: (0, 0)),
        pl.BlockSpec((1, 128), lambda i: (0, 0)),
    ],
    out_specs=pl.BlockSpec((BATCH, 128), lambda i: (0, 0)),
    out_shape=jax.ShapeDtypeStruct((BATCH, 128), _f32),
    scratch_shapes=[pltpu.VMEM((BATCH, D), _f32)],
)


def kernel(feats, edge_index, etype, emb0, emb1, emb2, emb3, W_rel, W_self,
           fcn_W1, fcn_b1, fcn_W2, fcn_b2):
    feats = feats.astype(_i32)
    src = edge_index[0].astype(_i32)
    dst = edge_index[1].astype(_i32)
    etype = etype.astype(_i32)

    flat = src * (2 * R) + etype * 2
    flat_p = jnp.pad(flat, (0, EPAD - E)).reshape(NS, G, CHUNK)
    flat2 = jnp.stack([flat_p, flat_p + 1])
    dst_p = jnp.pad(dst, (0, EPAD - E),
                    constant_values=N).reshape(NS, G, CHUNK)

    fidx = jnp.pad(feats.T, ((0, 0), (0, NPAD - N))).reshape(4, NW, GT, CHUNK)

    w_cat = jnp.concatenate(
        [W_rel.transpose(0, 2, 1, 3).reshape(NL, D, R * D), W_self], axis=-1)

    zeros_acc = jnp.zeros((ZROWS, DH), _f32)
    pool_mat = jnp.where(
        (jnp.arange(N, dtype=_i32)[:, None] // NPG)
        == jnp.arange(BATCH, dtype=_i32)[None, :],
        jnp.float32(1.0 / NPG), jnp.float32(0.0))
    w2p = jnp.pad(fcn_W2, ((0, 0), (0, 128 - 1)))
    b2p = jnp.pad(fcn_b2, (0, 128 - 1)).reshape(1, 128)
    b1r = fcn_b1.reshape(1, D)

    h0, h1, h2, h3 = _sc_embed(fidx, emb0, emb1, emb2, emb3)

    rel, selfo = _tc_layer0(h0[:N], h1[:N], h2[:N], h3[:N], w_cat[0])
    for l in range(NL):
        agg_lo, agg_hi = _sc_edges(
            rel.reshape(2 * R * N, DH), flat2, dst_p, zeros_acc)
        if l < NL - 1:
            rel, selfo = _tc_layer(agg_lo, agg_hi, selfo, w_cat[l + 1])

    out = _tc_head(agg_lo, agg_hi, selfo, pool_mat, fcn_W1, b1r, w2p, b2p)
    return out[:, :1]


# speedup vs baseline: 4.3738x; 1.0468x over previous
import functools

import jax
import jax.numpy as jnp
from jax import lax
from jax.experimental import pallas as pl
from jax.experimental.pallas import tpu as pltpu
from jax.experimental.pallas import tpu_sc as plsc

N = 50000
E = 800000
R = 8
D = 64
DH = 32
NL = 4
BATCH = 16
NPG = 3125

NC = 2
NS = 16
CHUNK = 128

G = (E + NS * CHUNK - 1) // (NS * CHUNK)
EPT = G * CHUNK
EPAD = NS * EPT
KB = 17
NBLK = G // KB

NW = NC * NS
GT = (N + NW * CHUNK - 1) // (NW * CHUNK)
NPT = GT * CHUNK
NPAD = NW * NPT

ACC_ROWS = NS * ((N + NS - 1) // NS + 1)
ZROWS = ACC_ROWS // NS
OPT = N // NS

BN = 2000
NB = N // BN

_f32 = jnp.float32
_i32 = jnp.int32


def _sc_embed_body(fidx, e0, e1, e2, e3, h0, h1, h2, h3, idx_v, rows_v,
                   sem_g0, sem_g1, sem_w0, sem_w1):
    c = lax.axis_index("c")
    s = lax.axis_index("s")
    w = s * NC + c
    sem_g = (sem_g0, sem_g1)
    sem_w = (sem_w0, sem_w1)
    steps = [(t, et, ht, g)
             for t, (et, ht) in enumerate(zip((e0, e1, e2, e3),
                                              (h0, h1, h2, h3)))
             for g in range(GT)]

    pltpu.sync_copy(fidx.at[0, w], idx_v.at[0])

    def gather(i):
        t, et, _, g = steps[i]
        pltpu.async_copy(et.at[idx_v.at[t % 2, g]], rows_v.at[i % 2],
                         sem_g[i % 2])

    def wait_gather(i):
        t, et, _, g = steps[i]
        pltpu.make_async_copy(et.at[idx_v.at[t % 2, g]], rows_v.at[i % 2],
                              sem_g[i % 2]).wait()

    def dst_ref(i):
        _, _, ht, g = steps[i]
        return ht.at[pl.ds(w * NPT + g * CHUNK, CHUNK)]

    gather(0)
    for i in range(len(steps)):
        t, _, _, g = steps[i]
        if g == GT - 1 and t < 3:
            pltpu.sync_copy(fidx.at[t + 1, w], idx_v.at[(t + 1) % 2])
        wait_gather(i)
        if i + 1 < len(steps):
            if i >= 1:
                pltpu.make_async_copy(rows_v.at[(i - 1) % 2],
                                      dst_ref(i - 1), sem_w[(i - 1) % 2]).wait()
            gather(i + 1)
        pltpu.async_copy(rows_v.at[i % 2], dst_ref(i), sem_w[i % 2])
    for i in (len(steps) - 2, len(steps) - 1):
        pltpu.make_async_copy(rows_v.at[i % 2], dst_ref(i),
                              sem_w[i % 2]).wait()


RING = 6
LOOKAHEAD = 3


def _sc_edges_body(rel_all, flat_i, dst_i, zeros_hbm, agg_lo, agg_hi,
                   flat_v, dst_v, rows_v, acc, *sems):
    c = lax.axis_index("c")
    s = lax.axis_index("s")
    sem_g, sem_s = sems[:RING], sems[RING:]
    pltpu.sync_copy(zeros_hbm, acc.at[pl.ds(s * ZROWS, ZROWS)])
    plsc.subcore_barrier()

    def gather(j):
        pltpu.async_copy(rel_all.at[flat_v.at[j]], rows_v.at[j % RING],
                         sem_g[j % RING])

    def wait_gather(j):
        pltpu.make_async_copy(rel_all.at[flat_v.at[j]], rows_v.at[j % RING],
                              sem_g[j % RING]).wait()

    def scatter(j):
        pltpu.async_copy(rows_v.at[j % RING], acc.at[dst_v.at[j]],
                         sem_s[j % RING], add=True)

    def wait_scatter(j):
        pltpu.make_async_copy(rows_v.at[j % RING], acc.at[dst_v.at[j]],
                              sem_s[j % RING]).wait()

    def outer(b, carry):
        pltpu.sync_copy(flat_i.at[c, s, pl.ds(b * KB, KB)], flat_v)
        pltpu.sync_copy(dst_i.at[s, pl.ds(b * KB, KB)], dst_v)
        for j in range(LOOKAHEAD):
            gather(j)
        for j in range(KB):
            wait_gather(j)
            nxt = j + LOOKAHEAD
            if nxt < KB:
                if j >= LOOKAHEAD:
                    wait_scatter(j - LOOKAHEAD)
                gather(nxt)
            scatter(j)
        for j in range(KB - 2 * LOOKAHEAD, KB):
            wait_scatter(j)
        return carry

    lax.fori_loop(0, NBLK, outer, 0)
    plsc.subcore_barrier()

    @pl.when(c == 0)
    def _():
        pltpu.sync_copy(acc.at[pl.ds(s * OPT, OPT)],
                        agg_lo.at[pl.ds(s * OPT, OPT)])

    @pl.when(c == 1)
    def _():
        pltpu.sync_copy(acc.at[pl.ds(s * OPT, OPT)],
                        agg_hi.at[pl.ds(s * OPT, OPT)])


@functools.lru_cache(maxsize=None)
def _sc_kernels():
    mesh = plsc.VectorSubcoreMesh(
        core_axis_name="c", subcore_axis_name="s",
        num_cores=NC, num_subcores=NS)
    params = pltpu.CompilerParams(use_tc_tiling_on_sc=False)
    embed = pl.kernel(
        _sc_embed_body,
        out_type=tuple(jax.ShapeDtypeStruct((NPAD, 16), _f32)
                       for _ in range(4)),
        mesh=mesh,
        scratch_types=[
            pltpu.VMEM((2, GT, CHUNK), _i32),
            pltpu.VMEM((2, CHUNK, 16), _f32),
            pltpu.SemaphoreType.DMA,
            pltpu.SemaphoreType.DMA,
            pltpu.SemaphoreType.DMA,
            pltpu.SemaphoreType.DMA,
        ],
        compiler_params=params)
    edges = pl.kernel(
        _sc_edges_body,
        out_type=(pltpu.HBM((N, DH), _f32),
                  pltpu.HBM((N, DH), _f32)),
        mesh=mesh,
        scratch_types=[
            pltpu.VMEM((KB, CHUNK), _i32),
            pltpu.VMEM((KB, CHUNK), _i32),
            pltpu.VMEM((RING, CHUNK, DH), _f32),
            pltpu.VMEM_SHARED((ACC_ROWS, DH), _f32),
        ] + [pltpu.SemaphoreType.DMA] * (2 * RING),
        compiler_params=params)
    return embed, edges


def _sc_embed(*args):
    return _sc_kernels()[0](*args)


def _sc_edges(*args):
    return _sc_kernels()[1](*args)


def _write_outs(o, rel_ref, self_ref):
    rel_ref[...] = o[:, :R * D]
    self_ref[...] = o[:, R * D:]


def _tc_layer0_body(h0_ref, h1_ref, h2_ref, h3_ref, w_ref, rel_ref, self_ref):
    w = w_ref[...]
    o = jnp.dot(h0_ref[...], w[0:16], preferred_element_type=_f32)
    for t, hr in enumerate((h1_ref, h2_ref, h3_ref), start=1):
        o += jnp.dot(hr[...], w[16 * t:16 * t + 16],
                     preferred_element_type=_f32)
    _write_outs(o, rel_ref, self_ref)


def _tc_layer_body(lo_in, hi_in, self_in, w_ref, rel_ref, self_ref):
    sp = self_in[...]
    hlo = jnp.maximum(lo_in[...] + sp[:, :DH], 0.0)
    hhi = jnp.maximum(hi_in[...] + sp[:, DH:], 0.0)
    w = w_ref[...]
    o = (jnp.dot(hlo, w[:DH], preferred_element_type=_f32)
         + jnp.dot(hhi, w[DH:], preferred_element_type=_f32))
    _write_outs(o, rel_ref, self_ref)


_LAYER_OUT = [
    jax.ShapeDtypeStruct((N, R * D), _f32),
    jax.ShapeDtypeStruct((N, D), _f32),
]
_LAYER_OUT_SPECS = [
    pl.BlockSpec((BN, R * D), lambda i: (i, 0)),
    pl.BlockSpec((BN, D), lambda i: (i, 0)),
]
_W_SPEC = pl.BlockSpec((D, R * D + D), lambda i: (0, 0))

_tc_layer0 = pl.pallas_call(
    _tc_layer0_body,
    grid=(NB,),
    in_specs=[pl.BlockSpec((BN, 16), lambda i: (i, 0)) for _ in range(4)]
    + [_W_SPEC],
    out_specs=_LAYER_OUT_SPECS,
    out_shape=_LAYER_OUT,
)

_tc_layer = pl.pallas_call(
    _tc_layer_body,
    grid=(NB,),
    in_specs=[
        pl.BlockSpec((BN, DH), lambda i: (i, 0)),
        pl.BlockSpec((BN, DH), lambda i: (i, 0)),
        pl.BlockSpec((BN, D), lambda i: (i, 0)),
        _W_SPEC,
    ],
    out_specs=_LAYER_OUT_SPECS,
    out_shape=_LAYER_OUT,
)


def _tc_head_body(lo_in, hi_in, self_in, pool_in, w1_ref, b1_ref, w2_ref,
                  b2_ref, out_ref, pooled):
    i = pl.program_id(0)

    @pl.when(i == 0)
    def _():
        pooled[...] = jnp.zeros_like(pooled)

    sp = self_in[...]
    hlo = jnp.maximum(lo_in[...] + sp[:, :DH], 0.0)
    hhi = jnp.maximum(hi_in[...] + sp[:, DH:], 0.0)
    pm = pool_in[...]
    dn = (((0,), (0,)), ((), ()))
    pooled[:, :DH] += lax.dot_general(pm, hlo, dn,
                                      preferred_element_type=_f32,
                                      precision=lax.Precision.HIGHEST)
    pooled[:, DH:] += lax.dot_general(pm, hhi, dn,
                                      preferred_element_type=_f32,
                                      precision=lax.Precision.HIGHEST)

    @pl.when(i == NB - 1)
    def _():
        p = pooled[...]
        hid = jnp.maximum(
            jnp.dot(p, w1_ref[...], preferred_element_type=_f32)
            + b1_ref[...], 0.0)
        out_ref[...] = (jnp.dot(hid, w2_ref[...],
                                preferred_element_type=_f32) + b2_ref[...])


_tc_head = pl.pallas_call(
    _tc_head_body,
    grid=(NB,),
    in_specs=[
        pl.BlockSpec((BN, DH), lambda i: (i, 0)),
        pl.BlockSpec((BN, DH), lambda i: (i, 0)),
        pl.BlockSpec((BN, D), lambda i: (i, 0)),
        pl.BlockSpec((BN, BATCH), lambda i: (i, 0)),
        pl.BlockSpec((D, D), lambda i: (0, 0)),
        pl.BlockSpec((1, D), lambda i: (0, 0)),
        pl.BlockSpec((D, 128), lambda i: (0, 0)),
        pl.BlockSpec((1, 128), lambda i: (0, 0)),
    ],
    out_specs=pl.BlockSpec((BATCH, 128), lambda i: (0, 0)),
    out_shape=jax.ShapeDtypeStruct((BATCH, 128), _f32),
    scratch_shapes=[pltpu.VMEM((BATCH, D), _f32)],
)


def kernel(feats, edge_index, etype, emb0, emb1, emb2, emb3, W_rel, W_self,
           fcn_W1, fcn_b1, fcn_W2, fcn_b2):
    feats = feats.astype(_i32)
    src = edge_index[0].astype(_i32)
    dst = edge_index[1].astype(_i32)
    etype = etype.astype(_i32)

    flat = src * (2 * R) + etype * 2
    flat_p = jnp.pad(flat, (0, EPAD - E)).reshape(NS, G, CHUNK)
    flat2 = jnp.stack([flat_p, flat_p + 1])
    dst_p = jnp.pad(dst, (0, EPAD - E),
                    constant_values=N).reshape(NS, G, CHUNK)

    fidx = jnp.pad(feats.T, ((0, 0), (0, NPAD - N))).reshape(4, NW, GT, CHUNK)

    w_cat = jnp.concatenate(
        [W_rel.transpose(0, 2, 1, 3).reshape(NL, D, R * D), W_self], axis=-1)

    zeros_acc = jnp.zeros((ZROWS, DH), _f32)
    pool_mat = jnp.where(
        (jnp.arange(N, dtype=_i32)[:, None] // NPG)
        == jnp.arange(BATCH, dtype=_i32)[None, :],
        jnp.float32(1.0 / NPG), jnp.float32(0.0))
    w2p = jnp.pad(fcn_W2, ((0, 0), (0, 128 - 1)))
    b2p = jnp.pad(fcn_b2, (0, 128 - 1)).reshape(1, 128)
    b1r = fcn_b1.reshape(1, D)

    h0, h1, h2, h3 = _sc_embed(fidx, emb0, emb1, emb2, emb3)

    rel, selfo = _tc_layer0(h0[:N], h1[:N], h2[:N], h3[:N], w_cat[0])
    for l in range(NL):
        agg_lo, agg_hi = _sc_edges(
            rel.reshape(2 * R * N, DH), flat2, dst_p, zeros_acc)
        if l < NL - 1:
            rel, selfo = _tc_layer(agg_lo, agg_hi, selfo, w_cat[l + 1])

    out = _tc_head(agg_lo, agg_hi, selfo, pool_mat, fcn_W1, b1r, w2p, b2p)
    return out[:, :1]


# speedup vs baseline: 4.3960x; 1.0051x over previous
import functools

import jax
import jax.numpy as jnp
from jax import lax
from jax.experimental import pallas as pl
from jax.experimental.pallas import tpu as pltpu
from jax.experimental.pallas import tpu_sc as plsc

N = 50000
E = 800000
R = 8
D = 64
DH = 32
NL = 4
BATCH = 16
NPG = 3125

NC = 2
NS = 16
CHUNK = 128

G = (E + NS * CHUNK - 1) // (NS * CHUNK)
EPT = G * CHUNK
EPAD = NS * EPT
KB = 17
NBLK = G // KB

NW = NC * NS
GT = (N + NW * CHUNK - 1) // (NW * CHUNK)
NPT = GT * CHUNK
NPAD = NW * NPT

ACC_ROWS = NS * ((N + NS - 1) // NS + 1)
ZROWS = ACC_ROWS // NS
OPT = N // NS

BN = 5000
NB = N // BN

_f32 = jnp.float32
_i32 = jnp.int32


def _sc_embed_body(fidx, e0, e1, e2, e3, h0, h1, h2, h3, idx_v, rows_v,
                   sem_g0, sem_g1, sem_w0, sem_w1):
    c = lax.axis_index("c")
    s = lax.axis_index("s")
    w = s * NC + c
    sem_g = (sem_g0, sem_g1)
    sem_w = (sem_w0, sem_w1)
    steps = [(t, et, ht, g)
             for t, (et, ht) in enumerate(zip((e0, e1, e2, e3),
                                              (h0, h1, h2, h3)))
             for g in range(GT)]

    pltpu.sync_copy(fidx.at[0, w], idx_v.at[0])

    def gather(i):
        t, et, _, g = steps[i]
        pltpu.async_copy(et.at[idx_v.at[t % 2, g]], rows_v.at[i % 2],
                         sem_g[i % 2])

    def wait_gather(i):
        t, et, _, g = steps[i]
        pltpu.make_async_copy(et.at[idx_v.at[t % 2, g]], rows_v.at[i % 2],
                              sem_g[i % 2]).wait()

    def dst_ref(i):
        _, _, ht, g = steps[i]
        return ht.at[pl.ds(w * NPT + g * CHUNK, CHUNK)]

    gather(0)
    for i in range(len(steps)):
        t, _, _, g = steps[i]
        if g == GT - 1 and t < 3:
            pltpu.sync_copy(fidx.at[t + 1, w], idx_v.at[(t + 1) % 2])
        wait_gather(i)
        if i + 1 < len(steps):
            if i >= 1:
                pltpu.make_async_copy(rows_v.at[(i - 1) % 2],
                                      dst_ref(i - 1), sem_w[(i - 1) % 2]).wait()
            gather(i + 1)
        pltpu.async_copy(rows_v.at[i % 2], dst_ref(i), sem_w[i % 2])
    for i in (len(steps) - 2, len(steps) - 1):
        pltpu.make_async_copy(rows_v.at[i % 2], dst_ref(i),
                              sem_w[i % 2]).wait()


RING = 6
LOOKAHEAD = 3


def _sc_edges_body(rel_all, flat_i, dst_i, zeros_hbm, agg_lo, agg_hi,
                   flat_v, dst_v, rows_v, acc, *sems):
    c = lax.axis_index("c")
    s = lax.axis_index("s")
    sem_g, sem_s = sems[:RING], sems[RING:]
    pltpu.sync_copy(zeros_hbm, acc.at[pl.ds(s * ZROWS, ZROWS)])
    plsc.subcore_barrier()

    def gather(j):
        pltpu.async_copy(rel_all.at[flat_v.at[j]], rows_v.at[j % RING],
                         sem_g[j % RING])

    def wait_gather(j):
        pltpu.make_async_copy(rel_all.at[flat_v.at[j]], rows_v.at[j % RING],
                              sem_g[j % RING]).wait()

    def scatter(j):
        pltpu.async_copy(rows_v.at[j % RING], acc.at[dst_v.at[j]],
                         sem_s[j % RING], add=True)

    def wait_scatter(j):
        pltpu.make_async_copy(rows_v.at[j % RING], acc.at[dst_v.at[j]],
                              sem_s[j % RING]).wait()

    def outer(b, carry):
        pltpu.sync_copy(flat_i.at[c, s, pl.ds(b * KB, KB)], flat_v)
        pltpu.sync_copy(dst_i.at[s, pl.ds(b * KB, KB)], dst_v)
        for j in range(LOOKAHEAD):
            gather(j)
        for j in range(KB):
            wait_gather(j)
            nxt = j + LOOKAHEAD
            if nxt < KB:
                if j >= LOOKAHEAD:
                    wait_scatter(j - LOOKAHEAD)
                gather(nxt)
            scatter(j)
        for j in range(KB - 2 * LOOKAHEAD, KB):
            wait_scatter(j)
        return carry

    lax.fori_loop(0, NBLK, outer, 0)
    plsc.subcore_barrier()

    @pl.when(c == 0)
    def _():
        pltpu.sync_copy(acc.at[pl.ds(s * OPT, OPT)],
                        agg_lo.at[pl.ds(s * OPT, OPT)])

    @pl.when(c == 1)
    def _():
        pltpu.sync_copy(acc.at[pl.ds(s * OPT, OPT)],
                        agg_hi.at[pl.ds(s * OPT, OPT)])


@functools.lru_cache(maxsize=None)
def _sc_kernels():
    mesh = plsc.VectorSubcoreMesh(
        core_axis_name="c", subcore_axis_name="s",
        num_cores=NC, num_subcores=NS)
    params = pltpu.CompilerParams(use_tc_tiling_on_sc=False)
    embed = pl.kernel(
        _sc_embed_body,
        out_type=tuple(jax.ShapeDtypeStruct((NPAD, 16), _f32)
                       for _ in range(4)),
        mesh=mesh,
        scratch_types=[
            pltpu.VMEM((2, GT, CHUNK), _i32),
            pltpu.VMEM((2, CHUNK, 16), _f32),
            pltpu.SemaphoreType.DMA,
            pltpu.SemaphoreType.DMA,
            pltpu.SemaphoreType.DMA,
            pltpu.SemaphoreType.DMA,
        ],
        compiler_params=params)
    edges = pl.kernel(
        _sc_edges_body,
        out_type=(pltpu.HBM((N, DH), _f32),
                  pltpu.HBM((N, DH), _f32)),
        mesh=mesh,
        scratch_types=[
            pltpu.VMEM((KB, CHUNK), _i32),
            pltpu.VMEM((KB, CHUNK), _i32),
            pltpu.VMEM((RING, CHUNK, DH), _f32),
            pltpu.VMEM_SHARED((ACC_ROWS, DH), _f32),
        ] + [pltpu.SemaphoreType.DMA] * (2 * RING),
        compiler_params=params)
    return embed, edges


def _sc_embed(*args):
    return _sc_kernels()[0](*args)


def _sc_edges(*args):
    return _sc_kernels()[1](*args)


def _write_outs(o, rel_ref, self_ref):
    rel_ref[...] = o[:, :R * D]
    self_ref[...] = o[:, R * D:]


def _tc_layer0_body(h0_ref, h1_ref, h2_ref, h3_ref, w_ref, rel_ref, self_ref):
    w = w_ref[...]
    o = jnp.dot(h0_ref[...], w[0:16], preferred_element_type=_f32)
    for t, hr in enumerate((h1_ref, h2_ref, h3_ref), start=1):
        o += jnp.dot(hr[...], w[16 * t:16 * t + 16],
                     preferred_element_type=_f32)
    _write_outs(o, rel_ref, self_ref)


def _tc_layer_body(lo_in, hi_in, self_in, w_ref, rel_ref, self_ref):
    sp = self_in[...]
    hlo = jnp.maximum(lo_in[...] + sp[:, :DH], 0.0)
    hhi = jnp.maximum(hi_in[...] + sp[:, DH:], 0.0)
    w = w_ref[...]
    o = (jnp.dot(hlo, w[:DH], preferred_element_type=_f32)
         + jnp.dot(hhi, w[DH:], preferred_element_type=_f32))
    _write_outs(o, rel_ref, self_ref)


_LAYER_OUT = [
    jax.ShapeDtypeStruct((N, R * D), _f32),
    jax.ShapeDtypeStruct((N, D), _f32),
]
_LAYER_OUT_SPECS = [
    pl.BlockSpec((BN, R * D), lambda i: (i, 0)),
    pl.BlockSpec((BN, D), lambda i: (i, 0)),
]
_W_SPEC = pl.BlockSpec((D, R * D + D), lambda i: (0, 0))

_tc_layer0 = pl.pallas_call(
    _tc_layer0_body,
    grid=(NB,),
    in_specs=[pl.BlockSpec((BN, 16), lambda i: (i, 0)) for _ in range(4)]
    + [_W_SPEC],
    out_specs=_LAYER_OUT_SPECS,
    out_shape=_LAYER_OUT,
)

_tc_layer = pl.pallas_call(
    _tc_layer_body,
    grid=(NB,),
    in_specs=[
        pl.BlockSpec((BN, DH), lambda i: (i, 0)),
        pl.BlockSpec((BN, DH), lambda i: (i, 0)),
        pl.BlockSpec((BN, D), lambda i: (i, 0)),
        _W_SPEC,
    ],
    out_specs=_LAYER_OUT_SPECS,
    out_shape=_LAYER_OUT,
)


def _tc_head_body(lo_in, hi_in, self_in, pool_in, w1_ref, b1_ref, w2_ref,
                  b2_ref, out_ref, pooled):
    i = pl.program_id(0)

    @pl.when(i == 0)
    def _():
        pooled[...] = jnp.zeros_like(pooled)

    sp = self_in[...]
    hlo = jnp.maximum(lo_in[...] + sp[:, :DH], 0.0)
    hhi = jnp.maximum(hi_in[...] + sp[:, DH:], 0.0)
    pm = pool_in[...]
    dn = (((0,), (0,)), ((), ()))
    pooled[:, :DH] += lax.dot_general(pm, hlo, dn,
                                      preferred_element_type=_f32,
                                      precision=lax.Precision.HIGHEST)
    pooled[:, DH:] += lax.dot_general(pm, hhi, dn,
                                      preferred_element_type=_f32,
                                      precision=lax.Precision.HIGHEST)

    @pl.when(i == NB - 1)
    def _():
        p = pooled[...]
        hid = jnp.maximum(
            jnp.dot(p, w1_ref[...], preferred_element_type=_f32)
            + b1_ref[...], 0.0)
        out_ref[...] = (jnp.dot(hid, w2_ref[...],
                                preferred_element_type=_f32) + b2_ref[...])


_tc_head = pl.pallas_call(
    _tc_head_body,
    grid=(NB,),
    in_specs=[
        pl.BlockSpec((BN, DH), lambda i: (i, 0)),
        pl.BlockSpec((BN, DH), lambda i: (i, 0)),
        pl.BlockSpec((BN, D), lambda i: (i, 0)),
        pl.BlockSpec((BN, BATCH), lambda i: (i, 0)),
        pl.BlockSpec((D, D), lambda i: (0, 0)),
        pl.BlockSpec((1, D), lambda i: (0, 0)),
        pl.BlockSpec((D, 128), lambda i: (0, 0)),
        pl.BlockSpec((1, 128), lambda i: (0, 0)),
    ],
    out_specs=pl.BlockSpec((BATCH, 128), lambda i: (0, 0)),
    out_shape=jax.ShapeDtypeStruct((BATCH, 128), _f32),
    scratch_shapes=[pltpu.VMEM((BATCH, D), _f32)],
)


def kernel(feats, edge_index, etype, emb0, emb1, emb2, emb3, W_rel, W_self,
           fcn_W1, fcn_b1, fcn_W2, fcn_b2):
    feats = feats.astype(_i32)
    src = edge_index[0].astype(_i32)
    dst = edge_index[1].astype(_i32)
    etype = etype.astype(_i32)

    flat = src * (2 * R) + etype * 2
    flat_p = jnp.pad(flat, (0, EPAD - E)).reshape(NS, G, CHUNK)
    flat2 = jnp.stack([flat_p, flat_p + 1])
    dst_p = jnp.pad(dst, (0, EPAD - E),
                    constant_values=N).reshape(NS, G, CHUNK)

    fidx = jnp.pad(feats.T, ((0, 0), (0, NPAD - N))).reshape(4, NW, GT, CHUNK)

    w_cat = jnp.concatenate(
        [W_rel.transpose(0, 2, 1, 3).reshape(NL, D, R * D), W_self], axis=-1)

    zeros_acc = jnp.zeros((ZROWS, DH), _f32)
    pool_mat = jnp.where(
        (jnp.arange(N, dtype=_i32)[:, None] // NPG)
        == jnp.arange(BATCH, dtype=_i32)[None, :],
        jnp.float32(1.0 / NPG), jnp.float32(0.0))
    w2p = jnp.pad(fcn_W2, ((0, 0), (0, 128 - 1)))
    b2p = jnp.pad(fcn_b2, (0, 128 - 1)).reshape(1, 128)
    b1r = fcn_b1.reshape(1, D)

    h0, h1, h2, h3 = _sc_embed(fidx, emb0, emb1, emb2, emb3)

    rel, selfo = _tc_layer0(h0[:N], h1[:N], h2[:N], h3[:N], w_cat[0])
    for l in range(NL):
        agg_lo, agg_hi = _sc_edges(
            rel.reshape(2 * R * N, DH), flat2, dst_p, zeros_acc)
        if l < NL - 1:
            rel, selfo = _tc_layer(agg_lo, agg_hi, selfo, w_cat[l + 1])

    out = _tc_head(agg_lo, agg_hi, selfo, pool_mat, fcn_W1, b1r, w2p, b2p)
    return out[:, :1]


# speedup vs baseline: 4.4474x; 1.0117x over previous
import functools

import jax
import jax.numpy as jnp
from jax import lax
from jax.experimental import pallas as pl
from jax.experimental.pallas import tpu as pltpu
from jax.experimental.pallas import tpu_sc as plsc

N = 50000
E = 800000
R = 8
D = 64
DH = 32
NL = 4
BATCH = 16
NPG = 3125

NC = 2
NS = 16
CHUNK = 128

G = (E + NS * CHUNK - 1) // (NS * CHUNK)
EPT = G * CHUNK
EPAD = NS * EPT
KB = 17
NBLK = G // KB

NW = NC * NS
GT = (N + NW * CHUNK - 1) // (NW * CHUNK)
NPT = GT * CHUNK
NPAD = NW * NPT

ACC_ROWS = NS * ((N + NS - 1) // NS + 1)
ZROWS = ACC_ROWS // NS
OPT = N // NS

BN = 5000
NB = N // BN

_f32 = jnp.float32
_i32 = jnp.int32


def _sc_embed_body(fidx, e0, e1, e2, e3, h0, h1, h2, h3, idx_v, rows_v,
                   sem_g0, sem_g1, sem_w0, sem_w1):
    c = lax.axis_index("c")
    s = lax.axis_index("s")
    w = s * NC + c
    sem_g = (sem_g0, sem_g1)
    sem_w = (sem_w0, sem_w1)
    steps = [(t, et, ht, g)
             for t, (et, ht) in enumerate(zip((e0, e1, e2, e3),
                                              (h0, h1, h2, h3)))
             for g in range(GT)]

    pltpu.sync_copy(fidx.at[0, w], idx_v.at[0])

    def gather(i):
        t, et, _, g = steps[i]
        pltpu.async_copy(et.at[idx_v.at[t % 2, g]], rows_v.at[i % 2],
                         sem_g[i % 2])

    def wait_gather(i):
        t, et, _, g = steps[i]
        pltpu.make_async_copy(et.at[idx_v.at[t % 2, g]], rows_v.at[i % 2],
                              sem_g[i % 2]).wait()

    def dst_ref(i):
        _, _, ht, g = steps[i]
        return ht.at[pl.ds(w * NPT + g * CHUNK, CHUNK)]

    gather(0)
    for i in range(len(steps)):
        t, _, _, g = steps[i]
        if g == GT - 1 and t < 3:
            pltpu.sync_copy(fidx.at[t + 1, w], idx_v.at[(t + 1) % 2])
        wait_gather(i)
        if i + 1 < len(steps):
            if i >= 1:
                pltpu.make_async_copy(rows_v.at[(i - 1) % 2],
                                      dst_ref(i - 1), sem_w[(i - 1) % 2]).wait()
            gather(i + 1)
        pltpu.async_copy(rows_v.at[i % 2], dst_ref(i), sem_w[i % 2])
    for i in (len(steps) - 2, len(steps) - 1):
        pltpu.make_async_copy(rows_v.at[i % 2], dst_ref(i),
                              sem_w[i % 2]).wait()


RING = 6
LOOKAHEAD = 3


def _sc_edges_body(rel_all, flat_i, dst_i, agg_lo, agg_hi,
                   flat_v, dst_v, rows_v, acc, *sems):
    c = lax.axis_index("c")
    s = lax.axis_index("s")
    sem_g, sem_s = sems[:RING], sems[RING:]

    zb = rows_v.at[0]
    zv = jnp.zeros((16,), _f32)

    def zfill(i, carry):
        zb[i, pl.ds(0, 16)] = zv
        zb[i, pl.ds(16, 16)] = zv
        return carry
    lax.fori_loop(0, CHUNK, zfill, 0)
    nfull = ZROWS // CHUNK
    rem = ZROWS - nfull * CHUNK

    def zcopy(i, carry):
        pltpu.sync_copy(zb, acc.at[pl.ds(s * ZROWS + i * CHUNK, CHUNK)])
        return carry
    lax.fori_loop(0, nfull, zcopy, 0)
    pltpu.sync_copy(zb.at[pl.ds(0, rem)],
                    acc.at[pl.ds(s * ZROWS + nfull * CHUNK, rem)])
    plsc.subcore_barrier()

    def gather(j):
        pltpu.async_copy(rel_all.at[flat_v.at[j]], rows_v.at[j % RING],
                         sem_g[j % RING])

    def wait_gather(j):
        pltpu.make_async_copy(rel_all.at[flat_v.at[j]], rows_v.at[j % RING],
                              sem_g[j % RING]).wait()

    def scatter(j):
        pltpu.async_copy(rows_v.at[j % RING], acc.at[dst_v.at[j]],
                         sem_s[j % RING], add=True)

    def wait_scatter(j):
        pltpu.make_async_copy(rows_v.at[j % RING], acc.at[dst_v.at[j]],
                              sem_s[j % RING]).wait()

    def outer(b, carry):
        pltpu.sync_copy(flat_i.at[c, s, pl.ds(b * KB, KB)], flat_v)
        pltpu.sync_copy(dst_i.at[s, pl.ds(b * KB, KB)], dst_v)
        for j in range(LOOKAHEAD):
            gather(j)
        for j in range(KB):
            wait_gather(j)
            nxt = j + LOOKAHEAD
            if nxt < KB:
                if j >= LOOKAHEAD:
                    wait_scatter(j - LOOKAHEAD)
                gather(nxt)
            scatter(j)
        for j in range(KB - 2 * LOOKAHEAD, KB):
            wait_scatter(j)
        return carry

    lax.fori_loop(0, NBLK, outer, 0)
    plsc.subcore_barrier()

    @pl.when(c == 0)
    def _():
        pltpu.sync_copy(acc.at[pl.ds(s * OPT, OPT)],
                        agg_lo.at[pl.ds(s * OPT, OPT)])

    @pl.when(c == 1)
    def _():
        pltpu.sync_copy(acc.at[pl.ds(s * OPT, OPT)],
                        agg_hi.at[pl.ds(s * OPT, OPT)])


@functools.lru_cache(maxsize=None)
def _sc_kernels():
    mesh = plsc.VectorSubcoreMesh(
        core_axis_name="c", subcore_axis_name="s",
        num_cores=NC, num_subcores=NS)
    params = pltpu.CompilerParams(use_tc_tiling_on_sc=False)
    embed = pl.kernel(
        _sc_embed_body,
        out_type=tuple(jax.ShapeDtypeStruct((NPAD, 16), _f32)
                       for _ in range(4)),
        mesh=mesh,
        scratch_types=[
            pltpu.VMEM((2, GT, CHUNK), _i32),
            pltpu.VMEM((2, CHUNK, 16), _f32),
            pltpu.SemaphoreType.DMA,
            pltpu.SemaphoreType.DMA,
            pltpu.SemaphoreType.DMA,
            pltpu.SemaphoreType.DMA,
        ],
        compiler_params=params)
    edges = pl.kernel(
        _sc_edges_body,
        out_type=(pltpu.HBM((N, DH), _f32),
                  pltpu.HBM((N, DH), _f32)),
        mesh=mesh,
        scratch_types=[
            pltpu.VMEM((KB, CHUNK), _i32),
            pltpu.VMEM((KB, CHUNK), _i32),
            pltpu.VMEM((RING, CHUNK, DH), _f32),
            pltpu.VMEM_SHARED((ACC_ROWS, DH), _f32),
        ] + [pltpu.SemaphoreType.DMA] * (2 * RING),
        compiler_params=params)
    return embed, edges


def _sc_embed(*args):
    return _sc_kernels()[0](*args)


def _sc_edges(*args):
    return _sc_kernels()[1](*args)


def _write_outs(o, rel_ref, self_ref):
    rel_ref[...] = o[:, :R * D]
    self_ref[...] = o[:, R * D:]


def _tc_layer0_body(h0_ref, h1_ref, h2_ref, h3_ref, w_ref, rel_ref, self_ref):
    w = w_ref[...]
    o = jnp.dot(h0_ref[...], w[0:16], preferred_element_type=_f32)
    for t, hr in enumerate((h1_ref, h2_ref, h3_ref), start=1):
        o += jnp.dot(hr[...], w[16 * t:16 * t + 16],
                     preferred_element_type=_f32)
    _write_outs(o, rel_ref, self_ref)


def _tc_layer_body(lo_in, hi_in, self_in, w_ref, rel_ref, self_ref):
    sp = self_in[...]
    hlo = jnp.maximum(lo_in[...] + sp[:, :DH], 0.0)
    hhi = jnp.maximum(hi_in[...] + sp[:, DH:], 0.0)
    w = w_ref[...]
    o = (jnp.dot(hlo, w[:DH], preferred_element_type=_f32)
         + jnp.dot(hhi, w[DH:], preferred_element_type=_f32))
    _write_outs(o, rel_ref, self_ref)


_LAYER_OUT = [
    jax.ShapeDtypeStruct((N, R * D), _f32),
    jax.ShapeDtypeStruct((N, D), _f32),
]
_LAYER_OUT_SPECS = [
    pl.BlockSpec((BN, R * D), lambda i: (i, 0)),
    pl.BlockSpec((BN, D), lambda i: (i, 0)),
]
_W_SPEC = pl.BlockSpec((D, R * D + D), lambda i: (0, 0))

_tc_layer0 = pl.pallas_call(
    _tc_layer0_body,
    grid=(NB,),
    in_specs=[pl.BlockSpec((BN, 16), lambda i: (i, 0)) for _ in range(4)]
    + [_W_SPEC],
    out_specs=_LAYER_OUT_SPECS,
    out_shape=_LAYER_OUT,
)

_tc_layer = pl.pallas_call(
    _tc_layer_body,
    grid=(NB,),
    in_specs=[
        pl.BlockSpec((BN, DH), lambda i: (i, 0)),
        pl.BlockSpec((BN, DH), lambda i: (i, 0)),
        pl.BlockSpec((BN, D), lambda i: (i, 0)),
        _W_SPEC,
    ],
    out_specs=_LAYER_OUT_SPECS,
    out_shape=_LAYER_OUT,
)


def _tc_head_body(lo_in, hi_in, self_in, pool_in, w1_ref, b1_ref, w2_ref,
                  b2_ref, out_ref, pooled):
    i = pl.program_id(0)

    @pl.when(i == 0)
    def _():
        pooled[...] = jnp.zeros_like(pooled)

    sp = self_in[...]
    hlo = jnp.maximum(lo_in[...] + sp[:, :DH], 0.0)
    hhi = jnp.maximum(hi_in[...] + sp[:, DH:], 0.0)
    pm = pool_in[...]
    dn = (((0,), (0,)), ((), ()))
    pooled[:, :DH] += lax.dot_general(pm, hlo, dn,
                                      preferred_element_type=_f32,
                                      precision=lax.Precision.HIGHEST)
    pooled[:, DH:] += lax.dot_general(pm, hhi, dn,
                                      preferred_element_type=_f32,
                                      precision=lax.Precision.HIGHEST)

    @pl.when(i == NB - 1)
    def _():
        p = pooled[...]
        hid = jnp.maximum(
            jnp.dot(p, w1_ref[...], preferred_element_type=_f32)
            + b1_ref[...], 0.0)
        out_ref[...] = (jnp.dot(hid, w2_ref[...],
                                preferred_element_type=_f32) + b2_ref[...])


_tc_head = pl.pallas_call(
    _tc_head_body,
    grid=(NB,),
    in_specs=[
        pl.BlockSpec((BN, DH), lambda i: (i, 0)),
        pl.BlockSpec((BN, DH), lambda i: (i, 0)),
        pl.BlockSpec((BN, D), lambda i: (i, 0)),
        pl.BlockSpec((BN, BATCH), lambda i: (i, 0)),
        pl.BlockSpec((D, D), lambda i: (0, 0)),
        pl.BlockSpec((1, D), lambda i: (0, 0)),
        pl.BlockSpec((D, 128), lambda i: (0, 0)),
        pl.BlockSpec((1, 128), lambda i: (0, 0)),
    ],
    out_specs=pl.BlockSpec((BATCH, 128), lambda i: (0, 0)),
    out_shape=jax.ShapeDtypeStruct((BATCH, 128), _f32),
    scratch_shapes=[pltpu.VMEM((BATCH, D), _f32)],
)


def kernel(feats, edge_index, etype, emb0, emb1, emb2, emb3, W_rel, W_self,
           fcn_W1, fcn_b1, fcn_W2, fcn_b2):
    feats = feats.astype(_i32)
    src = edge_index[0].astype(_i32)
    dst = edge_index[1].astype(_i32)
    etype = etype.astype(_i32)

    flat = src * (2 * R) + etype * 2
    flat_p = jnp.pad(flat, (0, EPAD - E)).reshape(NS, G, CHUNK)
    flat2 = jnp.stack([flat_p, flat_p + 1])
    dst_p = jnp.pad(dst, (0, EPAD - E),
                    constant_values=N).reshape(NS, G, CHUNK)

    fidx = jnp.pad(feats.T, ((0, 0), (0, NPAD - N))).reshape(4, NW, GT, CHUNK)

    w_cat = jnp.concatenate(
        [W_rel.transpose(0, 2, 1, 3).reshape(NL, D, R * D), W_self], axis=-1)

    pool_mat = jnp.where(
        (jnp.arange(N, dtype=_i32)[:, None] // NPG)
        == jnp.arange(BATCH, dtype=_i32)[None, :],
        jnp.float32(1.0 / NPG), jnp.float32(0.0))
    w2p = jnp.pad(fcn_W2, ((0, 0), (0, 128 - 1)))
    b2p = jnp.pad(fcn_b2, (0, 128 - 1)).reshape(1, 128)
    b1r = fcn_b1.reshape(1, D)

    h0, h1, h2, h3 = _sc_embed(fidx, emb0, emb1, emb2, emb3)

    rel, selfo = _tc_layer0(h0[:N], h1[:N], h2[:N], h3[:N], w_cat[0])
    for l in range(NL):
        agg_lo, agg_hi = _sc_edges(
            rel.reshape(2 * R * N, DH), flat2, dst_p)
        if l < NL - 1:
            rel, selfo = _tc_layer(agg_lo, agg_hi, selfo, w_cat[l + 1])

    out = _tc_head(agg_lo, agg_hi, selfo, pool_mat, fcn_W1, b1r, w2p, b2p)
    return out[:, :1]
